# baseline (speedup 1.0000x reference)
import functools

import jax
import jax.numpy as jnp
from jax.experimental import pallas as pl
from jax.experimental.pallas import tpu as pltpu

_TOTAL = 16384
_D = 512
_NB = 16
_BLK = 4096
_GRID = _TOTAL // _BLK


def _fused_body(x_ref, w_ref, benc_ref, wagg_ref, starts_ref, ends_ref,
                out_ref):
    i = pl.program_id(0)
    h = jnp.maximum(
        jnp.dot(x_ref[...], w_ref[...], preferred_element_type=jnp.float32)
        + benc_ref[...], 0.0)
    s = jnp.sum(h * wagg_ref[...], axis=1, keepdims=True)

    rows = i * _BLK + jax.lax.broadcasted_iota(jnp.int32, (_NB, _BLK), 1)
    mask = ((rows >= starts_ref[...]) & (rows < ends_ref[...])
            ).astype(jnp.float32)
    out_ref[...] = jnp.dot(mask, s, preferred_element_type=jnp.float32
                           ).reshape(1, _NB, 1)


def kernel(x, bag_sizes, W_enc, b_enc, W_agg, b_agg):
    starts = bag_sizes[:_NB].reshape(_NB, 1)
    ends = bag_sizes[1:].reshape(_NB, 1)
    benc = b_enc.reshape(1, _D)

    partials = pl.pallas_call(
        _fused_body,
        grid=(_GRID,),
        in_specs=[
            pl.BlockSpec((_BLK, _D), lambda i: (i, 0)),
            pl.BlockSpec((_D, _D), lambda i: (0, 0)),
            pl.BlockSpec((1, _D), lambda i: (0, 0)),
            pl.BlockSpec((1, _D), lambda i: (0, 0)),
            pl.BlockSpec((_NB, 1), lambda i: (0, 0)),
            pl.BlockSpec((_NB, 1), lambda i: (0, 0)),
        ],
        out_specs=pl.BlockSpec((1, _NB, 1), lambda i: (i, 0, 0)),
        out_shape=jax.ShapeDtypeStruct((_GRID, _NB, 1), jnp.float32),
        compiler_params=pltpu.CompilerParams(
            dimension_semantics=("parallel",)),
    )(x, W_enc, benc, W_agg.reshape(1, _D), starts, ends)

    counts = jnp.maximum((ends - starts).astype(jnp.float32), 1.0)
    return partials.sum(axis=0) / counts + b_agg.reshape(1, 1)
